# baseline (speedup 1.0000x reference)
import jax
import jax.numpy as jnp
from jax import lax
from jax.experimental import pallas as pl
from jax.experimental.pallas import tpu as pltpu

B = 2048
IN = 4096
SOMA = 1024
BR = 8
SAMP = 8
ND = SOMA * BR
SLOPE = 0.1

DBLK = 512
NB = ND // DBLK
SOMA_BLK = DBLK // BR


def _leaky(v):
    return jnp.where(v >= 0, v, SLOPE * v)


def _body(idx_ref, w_ref, bias_ref, cable_ref, sbias_ref, x_ref,
          out_d_ref, out_s_ref, s_scr):
    k = pl.program_id(0)

    @pl.when(k < NB)
    def _build():
        row_iota = lax.broadcasted_iota(jnp.int16, (IN, DBLK), 0)
        s_acc = jnp.zeros((IN, DBLK), jnp.bfloat16)
        for s in range(SAMP):
            idx_row = idx_ref[0, s, :][None, :]
            w_row = w_ref[0, s, :][None, :]
            hit = row_iota == idx_row
            s_acc = s_acc + jnp.where(hit, w_row, jnp.bfloat16(0.0))
        s_scr[k % 2] = s_acc

    @pl.when(k > 0)
    def _matmul():
        pre = jnp.dot(x_ref[...], s_scr[(k - 1) % 2],
                      preferred_element_type=jnp.float32)
        pre = pre + bias_ref[0]
        dact = _leaky(pre)
        out_d_ref[...] = dact

        d_iota = lax.broadcasted_iota(jnp.int32, (DBLK, SOMA_BLK), 0) >> 3
        m_iota = lax.broadcasted_iota(jnp.int32, (DBLK, SOMA_BLK), 1)
        c_mat = jnp.where(d_iota == m_iota, cable_ref[0], 0.0)
        c_mat = c_mat.astype(jnp.bfloat16)
        spre = jnp.dot(dact.astype(jnp.bfloat16), c_mat,
                       preferred_element_type=jnp.float32)
        spre = spre + sbias_ref[0]
        out_s_ref[0] = _leaky(spre)


def kernel(x, dendrite_indices, synaptic_weights, synaptic_bias,
           cable_weights, soma_bias):
    x_bf = x.astype(jnp.bfloat16)
    idx_t = dendrite_indices.astype(jnp.int16).T.reshape(SAMP, NB, DBLK)
    idx_t = jnp.transpose(idx_t, (1, 0, 2))
    w_t = synaptic_weights.astype(jnp.bfloat16).T.reshape(SAMP, NB, DBLK)
    w_t = jnp.transpose(w_t, (1, 0, 2))
    bias_r = synaptic_bias.reshape(NB, 1, DBLK)
    cable_b = jnp.broadcast_to(
        cable_weights.reshape(NB, DBLK, 1), (NB, DBLK, SOMA_BLK))
    sbias_r = soma_bias.reshape(NB, 1, SOMA_BLK)

    build = lambda k: (jnp.minimum(k, NB - 1), 0, 0)
    drain = lambda k: (jnp.maximum(k - 1, 0), 0, 0)

    grid = (NB + 1,)
    dact_flat, soma_act = pl.pallas_call(
        _body,
        grid=grid,
        in_specs=[
            pl.BlockSpec((1, SAMP, DBLK), build),
            pl.BlockSpec((1, SAMP, DBLK), build),
            pl.BlockSpec((1, 1, DBLK), drain),
            pl.BlockSpec((1, DBLK, SOMA_BLK), drain),
            pl.BlockSpec((1, 1, SOMA_BLK), drain),
            pl.BlockSpec((B, IN), lambda k: (0, 0)),
        ],
        out_specs=[
            pl.BlockSpec((B, DBLK), lambda k: (0, jnp.maximum(k - 1, 0))),
            pl.BlockSpec((1, B, SOMA_BLK), drain),
        ],
        out_shape=[
            jax.ShapeDtypeStruct((B, ND), jnp.float32),
            jax.ShapeDtypeStruct((NB, B, SOMA_BLK), jnp.float32),
        ],
        scratch_shapes=[pltpu.VMEM((2, IN, DBLK), jnp.bfloat16)],
    )(idx_t, w_t, bias_r, cable_b, sbias_r, x_bf)

    soma_out = jnp.transpose(soma_act, (1, 0, 2)).reshape(B, SOMA)
    return soma_out, dact_flat.reshape(B, SOMA, BR)


# speedup vs baseline: 1.1152x; 1.1152x over previous
import jax
import jax.numpy as jnp
from jax import lax
from jax.experimental import pallas as pl
from jax.experimental.pallas import tpu as pltpu

B = 2048
IN = 4096
SOMA = 1024
BR = 8
SAMP = 8
ND = SOMA * BR
SLOPE = 0.1

DBLK = 512
NB = ND // DBLK
SOMA_BLK = DBLK // BR


def _leaky(v):
    return jnp.where(v >= 0, v, SLOPE * v)


def _body(idx_ref, w_ref, bias_ref, cable_ref, sbias_ref, x_ref,
          out_d_ref, out_s_ref, s_scr):
    k = pl.program_id(0)

    pre = jnp.dot(x_ref[...], s_scr[(k + 1) % 2],
                  preferred_element_type=jnp.float32)
    pre = pre + bias_ref[0]
    dact = _leaky(pre)
    out_d_ref[...] = dact

    d_iota = lax.broadcasted_iota(jnp.int32, (DBLK, SOMA_BLK), 0) >> 3
    m_iota = lax.broadcasted_iota(jnp.int32, (DBLK, SOMA_BLK), 1)
    c_mat = jnp.where(d_iota == m_iota, cable_ref[0], 0.0)
    c_mat = c_mat.astype(jnp.bfloat16)
    spre = jnp.dot(dact.astype(jnp.bfloat16), c_mat,
                   preferred_element_type=jnp.float32)
    spre = spre + sbias_ref[0]
    out_s_ref[0] = _leaky(spre)

    row_iota = lax.broadcasted_iota(jnp.int16, (IN, DBLK), 0)
    s_acc = jnp.zeros((IN, DBLK), jnp.bfloat16)
    for s in range(SAMP):
        idx_row = idx_ref[0, s, :][None, :]
        w_row = w_ref[0, s, :][None, :]
        hit = row_iota == idx_row
        s_acc = s_acc + jnp.where(hit, w_row, jnp.bfloat16(0.0))
    s_scr[k % 2] = s_acc


def kernel(x, dendrite_indices, synaptic_weights, synaptic_bias,
           cable_weights, soma_bias):
    x_bf = x.astype(jnp.bfloat16)
    idx_t = dendrite_indices.astype(jnp.int16).T.reshape(SAMP, NB, DBLK)
    idx_t = jnp.transpose(idx_t, (1, 0, 2))
    w_t = synaptic_weights.astype(jnp.bfloat16).T.reshape(SAMP, NB, DBLK)
    w_t = jnp.transpose(w_t, (1, 0, 2))
    bias_r = synaptic_bias.reshape(NB, 1, DBLK)
    cable_b = jnp.broadcast_to(
        cable_weights.reshape(NB, DBLK, 1), (NB, DBLK, SOMA_BLK))
    sbias_r = soma_bias.reshape(NB, 1, SOMA_BLK)

    build = lambda k: (jnp.minimum(k, NB - 1), 0, 0)
    drain = lambda k: (jnp.maximum(k - 1, 0), 0, 0)

    grid = (NB + 1,)
    dact_flat, soma_act = pl.pallas_call(
        _body,
        grid=grid,
        in_specs=[
            pl.BlockSpec((1, SAMP, DBLK), build),
            pl.BlockSpec((1, SAMP, DBLK), build),
            pl.BlockSpec((1, 1, DBLK), drain),
            pl.BlockSpec((1, DBLK, SOMA_BLK), drain),
            pl.BlockSpec((1, 1, SOMA_BLK), drain),
            pl.BlockSpec((B, IN), lambda k: (0, 0)),
        ],
        out_specs=[
            pl.BlockSpec((B, DBLK), lambda k: (0, jnp.maximum(k - 1, 0))),
            pl.BlockSpec((1, B, SOMA_BLK), drain),
        ],
        out_shape=[
            jax.ShapeDtypeStruct((B, ND), jnp.float32),
            jax.ShapeDtypeStruct((NB, B, SOMA_BLK), jnp.float32),
        ],
        scratch_shapes=[pltpu.VMEM((2, IN, DBLK), jnp.bfloat16)],
    )(idx_t, w_t, bias_r, cable_b, sbias_r, x_bf)

    soma_out = jnp.transpose(soma_act, (1, 0, 2)).reshape(B, SOMA)
    return soma_out, dact_flat.reshape(B, SOMA, BR)


# speedup vs baseline: 1.1498x; 1.0311x over previous
import jax
import jax.numpy as jnp
from jax import lax
from jax.experimental import pallas as pl
from jax.experimental.pallas import tpu as pltpu

B = 2048
IN = 4096
SOMA = 1024
BR = 8
SAMP = 8
ND = SOMA * BR
SLOPE = 0.1

DBLK = 512
NB = ND // DBLK
SOMA_BLK = DBLK // BR


def _leaky(v):
    return jnp.where(v >= 0, v, SLOPE * v)


def _body(idx_ref, w_ref, bias_ref, cable_ref, sbias_ref, x_ref,
          out_d_ref, out_s_ref, s_scr):
    k = pl.program_id(0)

    pre = jnp.dot(x_ref[...], s_scr[(k + 1) % 2],
                  preferred_element_type=jnp.float32)
    pre = pre + bias_ref[0]
    dact = _leaky(pre)
    out_d_ref[...] = dact

    d_iota = lax.broadcasted_iota(jnp.int32, (DBLK, SOMA_BLK), 0) >> 3
    m_iota = lax.broadcasted_iota(jnp.int32, (DBLK, SOMA_BLK), 1)
    m01 = jnp.where(d_iota == m_iota, 1.0, 0.0).astype(jnp.bfloat16)
    scaled = (dact * cable_ref[0]).astype(jnp.bfloat16)
    spre = jnp.dot(scaled, m01, preferred_element_type=jnp.float32)
    spre = spre + sbias_ref[0]
    soma = _leaky(spre)

    first_of_pair = ((k - 1) % 2) == 0
    left = jnp.where(first_of_pair, soma, out_s_ref[:, :SOMA_BLK])
    out_s_ref[...] = jnp.concatenate([left, soma], axis=1)

    row_iota = lax.broadcasted_iota(jnp.int16, (IN, DBLK), 0)
    s_acc = jnp.zeros((IN, DBLK), jnp.bfloat16)
    for s in range(SAMP):
        idx_row = idx_ref[0, s, :][None, :]
        w_row = w_ref[0, s, :][None, :]
        hit = row_iota == idx_row
        s_acc = s_acc + jnp.where(hit, w_row, jnp.bfloat16(0.0))
    s_scr[k % 2] = s_acc


def kernel(x, dendrite_indices, synaptic_weights, synaptic_bias,
           cable_weights, soma_bias):
    x_bf = x.astype(jnp.bfloat16)
    idx_t = dendrite_indices.astype(jnp.int16).T.reshape(SAMP, NB, DBLK)
    idx_t = jnp.transpose(idx_t, (1, 0, 2))
    w_t = synaptic_weights.astype(jnp.bfloat16).T.reshape(SAMP, NB, DBLK)
    w_t = jnp.transpose(w_t, (1, 0, 2))
    bias_r = synaptic_bias.reshape(NB, 1, DBLK)
    cable_r = cable_weights.reshape(NB, 1, DBLK)
    sbias_r = soma_bias.reshape(NB, 1, SOMA_BLK)

    build = lambda k: (jnp.minimum(k, NB - 1), 0, 0)
    drain = lambda k: (jnp.maximum(k - 1, 0), 0, 0)

    grid = (NB + 1,)
    dact_flat, soma_act = pl.pallas_call(
        _body,
        grid=grid,
        in_specs=[
            pl.BlockSpec((1, SAMP, DBLK), build),
            pl.BlockSpec((1, SAMP, DBLK), build),
            pl.BlockSpec((1, 1, DBLK), drain),
            pl.BlockSpec((1, 1, DBLK), drain),
            pl.BlockSpec((1, 1, SOMA_BLK), drain),
            pl.BlockSpec((B, IN), lambda k: (0, 0)),
        ],
        out_specs=[
            pl.BlockSpec((B, DBLK), lambda k: (0, jnp.maximum(k - 1, 0))),
            pl.BlockSpec((B, 2 * SOMA_BLK),
                         lambda k: (0, jnp.maximum(k - 1, 0) // 2)),
        ],
        out_shape=[
            jax.ShapeDtypeStruct((B, ND), jnp.float32),
            jax.ShapeDtypeStruct((B, SOMA), jnp.float32),
        ],
        scratch_shapes=[pltpu.VMEM((2, IN, DBLK), jnp.bfloat16)],
    )(idx_t, w_t, bias_r, cable_r, sbias_r, x_bf)

    return soma_act, dact_flat.reshape(B, SOMA, BR)


# speedup vs baseline: 1.1509x; 1.0009x over previous
import jax
import jax.numpy as jnp
from jax import lax
from jax.experimental import pallas as pl
from jax.experimental.pallas import tpu as pltpu

B = 2048
IN = 4096
SOMA = 1024
BR = 8
SAMP = 8
ND = SOMA * BR
SLOPE = 0.1

DBLK = 512
NB = ND // DBLK
SOMA_BLK = DBLK // BR


def _leaky(v):
    return jnp.where(v >= 0, v, SLOPE * v)


def _cast_body(x_ref, o_ref):
    o_ref[...] = x_ref[...].astype(jnp.bfloat16)


def _cast_bf16(x):
    return pl.pallas_call(
        _cast_body,
        grid=(8,),
        in_specs=[pl.BlockSpec((B // 8, IN), lambda k: (k, 0))],
        out_specs=pl.BlockSpec((B // 8, IN), lambda k: (k, 0)),
        out_shape=jax.ShapeDtypeStruct((B, IN), jnp.bfloat16),
    )(x)


def _body(idx_ref, w_ref, bias_ref, cable_ref, sbias_ref, x_ref,
          out_d_ref, out_s_ref, s_scr):
    k = pl.program_id(0)

    pre = jnp.dot(x_ref[...], s_scr[(k + 1) % 2],
                  preferred_element_type=jnp.float32)
    pre = pre + bias_ref[0]
    dact = _leaky(pre)
    out_d_ref[...] = dact

    d_iota = lax.broadcasted_iota(jnp.int32, (DBLK, SOMA_BLK), 0) >> 3
    m_iota = lax.broadcasted_iota(jnp.int32, (DBLK, SOMA_BLK), 1)
    m01 = jnp.where(d_iota == m_iota, 1.0, 0.0).astype(jnp.bfloat16)
    scaled = (dact * cable_ref[0]).astype(jnp.bfloat16)
    spre = jnp.dot(scaled, m01, preferred_element_type=jnp.float32)
    spre = spre + sbias_ref[0]
    soma = _leaky(spre)

    first_of_pair = ((k - 1) % 2) == 0
    left = jnp.where(first_of_pair, soma, out_s_ref[:, :SOMA_BLK])
    out_s_ref[...] = jnp.concatenate([left, soma], axis=1)

    row_iota = lax.broadcasted_iota(jnp.int16, (IN, DBLK), 0)
    s_acc = jnp.zeros((IN, DBLK), jnp.bfloat16)
    for s in range(SAMP):
        idx_row = idx_ref[0, s, :][None, :]
        w_row = w_ref[0, s, :][None, :]
        hit = row_iota == idx_row
        s_acc = s_acc + jnp.where(hit, w_row, jnp.bfloat16(0.0))
    s_scr[k % 2] = s_acc


def kernel(x, dendrite_indices, synaptic_weights, synaptic_bias,
           cable_weights, soma_bias):
    x_bf = _cast_bf16(x)
    idx_t = dendrite_indices.astype(jnp.int16).T.reshape(SAMP, NB, DBLK)
    idx_t = jnp.transpose(idx_t, (1, 0, 2))
    w_t = synaptic_weights.astype(jnp.bfloat16).T.reshape(SAMP, NB, DBLK)
    w_t = jnp.transpose(w_t, (1, 0, 2))
    bias_r = synaptic_bias.reshape(NB, 1, DBLK)
    cable_r = cable_weights.reshape(NB, 1, DBLK)
    sbias_r = soma_bias.reshape(NB, 1, SOMA_BLK)

    build = lambda k: (jnp.minimum(k, NB - 1), 0, 0)
    drain = lambda k: (jnp.maximum(k - 1, 0), 0, 0)

    grid = (NB + 1,)
    dact_flat, soma_act = pl.pallas_call(
        _body,
        grid=grid,
        in_specs=[
            pl.BlockSpec((1, SAMP, DBLK), build),
            pl.BlockSpec((1, SAMP, DBLK), build),
            pl.BlockSpec((1, 1, DBLK), drain),
            pl.BlockSpec((1, 1, DBLK), drain),
            pl.BlockSpec((1, 1, SOMA_BLK), drain),
            pl.BlockSpec((B, IN), lambda k: (0, 0)),
        ],
        out_specs=[
            pl.BlockSpec((B, DBLK), lambda k: (0, jnp.maximum(k - 1, 0))),
            pl.BlockSpec((B, 2 * SOMA_BLK),
                         lambda k: (0, jnp.maximum(k - 1, 0) // 2)),
        ],
        out_shape=[
            jax.ShapeDtypeStruct((B, ND), jnp.float32),
            jax.ShapeDtypeStruct((B, SOMA), jnp.float32),
        ],
        scratch_shapes=[pltpu.VMEM((2, IN, DBLK), jnp.bfloat16)],
    )(idx_t, w_t, bias_r, cable_r, sbias_r, x_bf)

    return soma_act, dact_flat.reshape(B, SOMA, BR)


# speedup vs baseline: 1.1618x; 1.0095x over previous
import jax
import jax.numpy as jnp
from jax import lax
from jax.experimental import pallas as pl
from jax.experimental.pallas import tpu as pltpu

B = 2048
IN = 4096
SOMA = 1024
BR = 8
SAMP = 8
ND = SOMA * BR
SLOPE = 0.1

DBLK = 512
NB = ND // DBLK
SOMA_BLK = DBLK // BR


def _leaky(v):
    return jnp.maximum(v, SLOPE * v)


def _dedup(idx, w):
    cols_w = [w[:, s] for s in range(SAMP)]
    for s in range(1, SAMP):
        taken = jnp.zeros((ND,), jnp.bool_)
        for e in range(s):
            m = (idx[:, e] == idx[:, s]) & ~taken
            cols_w[e] = cols_w[e] + jnp.where(m, cols_w[s], 0.0)
            taken = taken | m
        cols_w[s] = jnp.where(taken, 0.0, cols_w[s])
    return jnp.stack(cols_w, axis=1)


def _cast_body(x_ref, o_ref):
    o_ref[...] = x_ref[...].astype(jnp.bfloat16)


def _cast_bf16(x):
    return pl.pallas_call(
        _cast_body,
        grid=(8,),
        in_specs=[pl.BlockSpec((B // 8, IN), lambda k: (k, 0))],
        out_specs=pl.BlockSpec((B // 8, IN), lambda k: (k, 0)),
        out_shape=jax.ShapeDtypeStruct((B, IN), jnp.bfloat16),
    )(x)


def _body(idx_ref, w_ref, bias_ref, cable_ref, sbias_ref, x_ref,
          out_d_ref, out_s_ref, s_scr):
    k = pl.program_id(0)

    pre = jnp.dot(x_ref[...], s_scr[(k + 1) % 2],
                  preferred_element_type=jnp.float32)
    pre = pre + bias_ref[0]
    dact = _leaky(pre)
    out_d_ref[...] = dact

    d_iota = lax.broadcasted_iota(jnp.int32, (DBLK, SOMA_BLK), 0) >> 3
    m_iota = lax.broadcasted_iota(jnp.int32, (DBLK, SOMA_BLK), 1)
    m01 = jnp.where(d_iota == m_iota, 1.0, 0.0).astype(jnp.bfloat16)
    scaled = (dact * cable_ref[0]).astype(jnp.bfloat16)
    spre = jnp.dot(scaled, m01, preferred_element_type=jnp.float32)
    spre = spre + sbias_ref[0]
    soma = _leaky(spre)

    first_of_pair = ((k - 1) % 2) == 0
    left = jnp.where(first_of_pair, soma, out_s_ref[:, :SOMA_BLK])
    out_s_ref[...] = jnp.concatenate([left, soma], axis=1)

    row_iota = lax.broadcasted_iota(jnp.int16, (IN, DBLK), 0)
    s_acc = jnp.zeros((IN, DBLK), jnp.bfloat16)
    for s in range(SAMP):
        idx_row = idx_ref[0, s, :][None, :]
        w_row = w_ref[0, s, :][None, :]
        hit = row_iota == idx_row
        s_acc = s_acc + jnp.where(hit, w_row, jnp.bfloat16(0.0))
    s_scr[k % 2] = s_acc


def kernel(x, dendrite_indices, synaptic_weights, synaptic_bias,
           cable_weights, soma_bias):
    x_bf = _cast_bf16(x)
    idx_t = dendrite_indices.astype(jnp.int16).T.reshape(SAMP, NB, DBLK)
    idx_t = jnp.transpose(idx_t, (1, 0, 2))
    w_t = synaptic_weights.astype(jnp.bfloat16).T.reshape(SAMP, NB, DBLK)
    w_t = jnp.transpose(w_t, (1, 0, 2))
    bias_r = synaptic_bias.reshape(NB, 1, DBLK)
    cable_r = cable_weights.reshape(NB, 1, DBLK)
    sbias_r = soma_bias.reshape(NB, 1, SOMA_BLK)

    build = lambda k: (jnp.minimum(k, NB - 1), 0, 0)
    drain = lambda k: (jnp.maximum(k - 1, 0), 0, 0)

    grid = (NB + 1,)
    dact_flat, soma_act = pl.pallas_call(
        _body,
        grid=grid,
        in_specs=[
            pl.BlockSpec((1, SAMP, DBLK), build),
            pl.BlockSpec((1, SAMP, DBLK), build),
            pl.BlockSpec((1, 1, DBLK), drain),
            pl.BlockSpec((1, 1, DBLK), drain),
            pl.BlockSpec((1, 1, SOMA_BLK), drain),
            pl.BlockSpec((B, IN), lambda k: (0, 0)),
        ],
        out_specs=[
            pl.BlockSpec((B, DBLK), lambda k: (0, jnp.maximum(k - 1, 0))),
            pl.BlockSpec((B, 2 * SOMA_BLK),
                         lambda k: (0, jnp.maximum(k - 1, 0) // 2)),
        ],
        out_shape=[
            jax.ShapeDtypeStruct((B, ND), jnp.float32),
            jax.ShapeDtypeStruct((B, SOMA), jnp.float32),
        ],
        scratch_shapes=[pltpu.VMEM((2, IN, DBLK), jnp.bfloat16)],
    )(idx_t, w_t, bias_r, cable_r, sbias_r, x_bf)

    return soma_act, dact_flat.reshape(B, SOMA, BR)


# speedup vs baseline: 1.1656x; 1.0033x over previous
import jax
import jax.numpy as jnp
from jax import lax
from jax.experimental import pallas as pl
from jax.experimental.pallas import tpu as pltpu

B = 2048
IN = 4096
SOMA = 1024
BR = 8
SAMP = 8
ND = SOMA * BR
SLOPE = 0.1

DBLK = 512
NB = ND // DBLK
SOMA_BLK = DBLK // BR


def _leaky(v):
    return jnp.maximum(v, SLOPE * v)


def _cast_body(x_ref, o_ref):
    o_ref[...] = x_ref[...].astype(jnp.bfloat16)


def _cast_bf16(x):
    return pl.pallas_call(
        _cast_body,
        grid=(8,),
        in_specs=[pl.BlockSpec((B // 8, IN), lambda k: (k, 0))],
        out_specs=pl.BlockSpec((B // 8, IN), lambda k: (k, 0)),
        out_shape=jax.ShapeDtypeStruct((B, IN), jnp.bfloat16),
    )(x)


def _body(idx_ref, w_ref, bias_ref, cable_ref, sbias_ref, x_ref,
          out_d_ref, out_s_ref, s_scr):
    k = pl.program_id(0)

    pre = jnp.dot(x_ref[...], s_scr[(k + 1) % 2],
                  preferred_element_type=jnp.float32)
    pre = pre + bias_ref[0]
    dact = _leaky(pre)
    out_d_ref[...] = dact

    d_iota = lax.broadcasted_iota(jnp.int32, (DBLK, SOMA_BLK), 0) >> 3
    m_iota = lax.broadcasted_iota(jnp.int32, (DBLK, SOMA_BLK), 1)
    m01 = jnp.where(d_iota == m_iota, 1.0, 0.0).astype(jnp.bfloat16)
    scaled = (dact * cable_ref[0]).astype(jnp.bfloat16)
    spre = jnp.dot(scaled, m01, preferred_element_type=jnp.float32)
    spre = spre + sbias_ref[0]
    soma = _leaky(spre)

    first_of_pair = ((k - 1) % 2) == 0
    left = jnp.where(first_of_pair, soma, out_s_ref[:, :SOMA_BLK])
    out_s_ref[...] = jnp.concatenate([left, soma], axis=1)

    idx_rows = [idx_ref[0, s, :][None, :] for s in range(SAMP)]
    w_rows = [w_ref[0, s, :][None, :] for s in range(SAMP)]
    for s in range(1, SAMP):
        taken = jnp.zeros_like(idx_rows[0], jnp.bool_)
        for e in range(s):
            m = (idx_rows[e] == idx_rows[s]) & ~taken
            w_rows[e] = w_rows[e] + jnp.where(m, w_rows[s],
                                              jnp.bfloat16(0.0))
            taken = taken | m
        w_rows[s] = jnp.where(taken, jnp.bfloat16(0.0), w_rows[s])

    row_iota = lax.broadcasted_iota(jnp.int16, (IN, DBLK), 0)
    s_acc = jnp.zeros((IN, DBLK), jnp.bfloat16)
    for s in reversed(range(SAMP)):
        hit = row_iota == idx_rows[s]
        s_acc = jnp.where(hit, w_rows[s], s_acc)
    s_scr[k % 2] = s_acc


def kernel(x, dendrite_indices, synaptic_weights, synaptic_bias,
           cable_weights, soma_bias):
    x_bf = _cast_bf16(x)
    idx_t = dendrite_indices.astype(jnp.int16).T.reshape(SAMP, NB, DBLK)
    idx_t = jnp.transpose(idx_t, (1, 0, 2))
    w_t = synaptic_weights.astype(jnp.bfloat16).T.reshape(SAMP, NB, DBLK)
    w_t = jnp.transpose(w_t, (1, 0, 2))
    bias_r = synaptic_bias.reshape(NB, 1, DBLK)
    cable_r = cable_weights.reshape(NB, 1, DBLK)
    sbias_r = soma_bias.reshape(NB, 1, SOMA_BLK)

    build = lambda k: (jnp.minimum(k, NB - 1), 0, 0)
    drain = lambda k: (jnp.maximum(k - 1, 0), 0, 0)

    grid = (NB + 1,)
    dact_flat, soma_act = pl.pallas_call(
        _body,
        grid=grid,
        in_specs=[
            pl.BlockSpec((1, SAMP, DBLK), build),
            pl.BlockSpec((1, SAMP, DBLK), build),
            pl.BlockSpec((1, 1, DBLK), drain),
            pl.BlockSpec((1, 1, DBLK), drain),
            pl.BlockSpec((1, 1, SOMA_BLK), drain),
            pl.BlockSpec((B, IN), lambda k: (0, 0)),
        ],
        out_specs=[
            pl.BlockSpec((B, DBLK), lambda k: (0, jnp.maximum(k - 1, 0))),
            pl.BlockSpec((B, 2 * SOMA_BLK),
                         lambda k: (0, jnp.maximum(k - 1, 0) // 2)),
        ],
        out_shape=[
            jax.ShapeDtypeStruct((B, ND), jnp.float32),
            jax.ShapeDtypeStruct((B, SOMA), jnp.float32),
        ],
        scratch_shapes=[pltpu.VMEM((2, IN, DBLK), jnp.bfloat16)],
    )(idx_t, w_t, bias_r, cable_r, sbias_r, x_bf)

    return soma_act, dact_flat.reshape(B, SOMA, BR)
